# Initial kernel scaffold; baseline (speedup 1.0000x reference)
import jax
import jax.numpy as jnp
from jax.experimental import pallas as pl


def kernel(x, edge_index, edge_attr, batch, params):
    raise NotImplementedError("write your pallas kernel here")


# speedup vs baseline: 6.8047x; 6.8047x over previous
import functools

import jax
import jax.numpy as jnp
import numpy as np
from jax import lax
from jax.experimental import pallas as pl
from jax.experimental.pallas import tpu as pltpu
from jax.experimental.pallas import tpu_sc as plsc

N = 10000
E = 320000
H = 64
NUM_GRAPHS = 128
_DEG_HIST = np.array([0.0] * 28 + [500.0, 1000.0, 2000.0, 3000.0, 2000.0, 1000.0, 500.0])
AVG_DEG_LOG = float((np.log(np.arange(len(_DEG_HIST)) + 1.0) * _DEG_HIST).sum() / _DEG_HIST.sum())

NW = 32
NPT = 320
NPAD = NW * NPT
CK = 128
NCHUNK = E // CK

f32 = jnp.float32
i32 = jnp.int32


def _bf(t):
    return t.astype(jnp.bfloat16).astype(f32)


def _dotd(a, b):
    return jnp.dot(_bf(a), _bf(b), preferred_element_type=f32,
                   precision=jax.lax.Precision.HIGHEST)


def _make_sc_edge():
    mesh = plsc.VectorSubcoreMesh(core_axis_name="c", subcore_axis_name="s")
    out = jax.ShapeDtypeStruct((NPAD // 2, 2 * H), f32)

    @functools.partial(
        pl.kernel,
        out_type=[out, out, out, out],
        mesh=mesh,
        scratch_types=[
            pltpu.VMEM((CK + 16,), i32),
            pltpu.VMEM((CK,), i32),
            pltpu.VMEM((CK, 2 * H), f32),
            pltpu.VMEM((16, H), f32),
            pltpu.VMEM((4 * NW,), i32),
            pltpu.VMEM((NPT // 2, 2 * H), f32),
            pltpu.VMEM((NPT // 2, 2 * H), f32),
            pltpu.VMEM((NPT // 2, 2 * H), f32),
            pltpu.VMEM((NPT // 2, 2 * H), f32),
            pltpu.SemaphoreType.DMA,
        ],
    )
    def sc_edge(keys_hbm, bt_hbm, ctab_hbm, meta_hbm, selfv_hbm, selfv2_hbm,
                s1_out, s2_out, mn_out, mx_out,
                keys_v, srcs_v, rows_v, ctab_v, meta_v, s1, s2, mn, mx, sem):
        wid = lax.axis_index("s") * 2 + lax.axis_index("c")
        nt0 = wid * NPT
        pltpu.sync_copy(meta_hbm, meta_v)
        pltpu.sync_copy(ctab_hbm, ctab_v)
        hp0 = wid * (NPT // 2)
        pltpu.sync_copy(selfv_hbm.at[pl.ds(hp0, NPT // 2)], s1)
        pltpu.sync_copy(selfv2_hbm.at[pl.ds(hp0, NPT // 2)], s2)
        pltpu.sync_copy(selfv_hbm.at[pl.ds(hp0, NPT // 2)], mn)
        pltpu.sync_copy(selfv_hbm.at[pl.ds(hp0, NPT // 2)], mx)
        lo = meta_v[pl.ds(wid, 16)][0]
        hi = meta_v[pl.ds(NW + wid, 16)][0]

        def chunk_body(ci, carry):
            base = ci * CK
            pltpu.sync_copy(keys_hbm.at[pl.ds(base, CK)], keys_v.at[pl.ds(0, CK)])
            for q in range(CK // 16):
                kk = keys_v[pl.ds(q * 16, 16)]
                srcs_v[pl.ds(q * 16, 16)] = (kk >> 4) & 8191
            pltpu.async_copy(bt_hbm.at[srcs_v], rows_v, sem).wait()

            def edge_body(j, carry2):
                kj = keys_v[pl.ds(j, 16)][0]
                d = (kj >> 17) - nt0
                cj = kj & 7
                off = ((kj >> 3) & 1) * H

                @pl.when((d >= 0) & (d < NPT))
                def _():
                    dr = d >> 1
                    doff = (d & 1) * H
                    for f in range(H // 16):
                        sl = pl.ds(doff + f * 16, 16)
                        v = rows_v[j, pl.ds(off + f * 16, 16)] + ctab_v[cj, pl.ds(f * 16, 16)]
                        s1[dr, sl] = s1[dr, sl] + v
                        s2[dr, sl] = s2[dr, sl] + v * v
                        mn[dr, sl] = jnp.minimum(mn[dr, sl], v)
                        mx[dr, sl] = jnp.maximum(mx[dr, sl], v)

                return carry2

            lax.fori_loop(0, CK, edge_body, jnp.int32(0))
            return carry

        lax.fori_loop(lo, hi, chunk_body, jnp.int32(0))
        pltpu.sync_copy(s1, s1_out.at[pl.ds(hp0, NPT // 2)])
        pltpu.sync_copy(s2, s2_out.at[pl.ds(hp0, NPT // 2)])
        pltpu.sync_copy(mn, mn_out.at[pl.ds(hp0, NPT // 2)])
        pltpu.sync_copy(mx, mx_out.at[pl.ds(hp0, NPT // 2)])

    return sc_edge


_sc_edge = _make_sc_edge()


_RB = 2560


def _enc_body(xf_ref, d_ref, base_ref, h_ref):
    h_ref[...] = jnp.dot(xf_ref[...], d_ref[...],
                         preferred_element_type=f32, precision=jax.lax.Precision.HIGHEST) + base_ref[...]


def _enc(xf, dmat, base):
    return pl.pallas_call(
        _enc_body,
        out_shape=jax.ShapeDtypeStruct((NPAD, H), f32),
    )(xf, dmat, base)


def _pre_body(h_ref, wd_ref, ws_ref, wedge_ref, we_ref, bpre_ref, bedge_ref,
              etab_ref, a_ref, b_ref, sv_ref, sv2_ref, ctab_ref):
    eetab = _dotd(etab_ref[...], wedge_ref[...]) + bedge_ref[...]
    ctab = _dotd(eetab, we_ref[...]) + bpre_ref[...]
    h = h_ref[...]
    a = _dotd(h, wd_ref[...])
    b = _dotd(h, ws_ref[...])
    sv = b + ctab[0:1, :]
    a_ref[...] = a
    b_ref[...] = b
    sv_ref[...] = sv
    sv2_ref[...] = sv * sv
    ctab_ref[...] = ctab


_PRE_C = 2


def _pre(h_pad, wd, ws, wedge, we, bpre, bedge, etab):
    ch = NPAD // _PRE_C
    o = jax.ShapeDtypeStruct((ch, H), f32)
    outs = []
    for i in range(_PRE_C):
        outs.append(pl.pallas_call(
            _pre_body,
            out_shape=[o, o, o, o, jax.ShapeDtypeStruct((16, H), f32)],
        )(h_pad[i * ch:(i + 1) * ch], wd, ws, wedge, we, bpre, bedge, etab))
    cat = lambda k: jnp.concatenate([t[k] for t in outs], axis=0)
    return cat(0), cat(1), cat(2), cat(3), outs[0][4]


def _cnt_body(plo_ref, phi_ref, inv_ref, sc1_ref, sc2_ref):
    cnt = (phi_ref[...] - plo_ref[...]).astype(f32) + 1.0
    inv_ref[...] = 1.0 / cnt
    degl = jnp.log(cnt + 1.0)
    sc1_ref[...] = degl * (1.0 / AVG_DEG_LOG)
    sc2_ref[...] = AVG_DEG_LOG / degl


def _cntk(plo_rs, phi_rs):
    o = jax.ShapeDtypeStruct((NPAD // H, H), f32)
    return pl.pallas_call(
        _cnt_body,
        out_shape=[o, o, o],
    )(plo_rs, phi_rs)


def _comb_body(h_ref, a_ref, s1_ref, s2_ref, mn_ref, mx_ref, inv_ref,
               sc1_ref, sc2_ref, wpost_ref, bpost_ref, wlin_ref, blin_ref,
               out_ref):
    inv = inv_ref[...]
    a = a_ref[...]
    s1 = s1_ref[...] * inv
    mean = a + s1
    var = s2_ref[...] * inv - s1 * s1
    std = jnp.sqrt(jnp.maximum(var, 0.0) + 1e-5)
    mn = a + mn_ref[...]
    mx = a + mx_ref[...]
    sc1 = sc1_ref[...]
    sc2 = sc2_ref[...]
    w = wpost_ref
    parts = (h_ref[...], mean, mn, mx, std,
             mean * sc1, mn * sc1, mx * sc1, std * sc1,
             mean * sc2, mn * sc2, mx * sc2, std * sc2)
    acc = bpost_ref[...]
    for k, p in enumerate(parts):
        acc = acc + _dotd(p, w[k * H:(k + 1) * H, :])
    out_ref[...] = _dotd(acc, wlin_ref[...]) + blin_ref[...]


_COMB_C = 4


def _comb(h_pad, a, s1, s2, mn, mx, invb, sc1b, sc2b, wpost, bpost, wlin, blin):
    ch = NPAD // _COMB_C
    outs = []
    for i in range(_COMB_C):
        sl = slice(i * ch, (i + 1) * ch)
        outs.append(pl.pallas_call(
            _comb_body,
            out_shape=jax.ShapeDtypeStruct((ch, H), f32),
        )(h_pad[sl], a[sl], s1[sl], s2[sl], mn[sl], mx[sl], invb[sl],
          sc1b[sl], sc2b[sl], wpost, bpost, wlin, blin))
    return jnp.concatenate(outs, axis=0)


def _bn_body(x_ref, g_ref, b_ref, o_ref):
    x = x_ref[...]
    mu = jnp.mean(x, axis=0, keepdims=True)
    var = jnp.mean(x * x, axis=0, keepdims=True) - mu * mu
    y = g_ref[...] * (x - mu) / jnp.sqrt(var + 1e-5) + b_ref[...]
    o_ref[...] = jnp.maximum(y, 0.0)


def _bn_relu(x, g, b):
    n = x.shape[0]
    return pl.pallas_call(
        _bn_body,
        out_shape=jax.ShapeDtypeStruct((n, H), f32),
    )(x, g, b)


def _pool_body(oh_ref, h_ref, g_ref):
    g_ref[...] = lax.dot_general(oh_ref[...], h_ref[...], (((0,), (0,)), ((), ())),
                                 preferred_element_type=f32, precision=jax.lax.Precision.HIGHEST)


def _pool(onehot, h):
    return pl.pallas_call(
        _pool_body,
        out_shape=jax.ShapeDtypeStruct((NUM_GRAPHS, H), f32),
    )(onehot, h)


def _mlp_body(g_ref, w1_ref, b1_ref, g1_ref, be1_ref, w2_ref, b2_ref, g2_ref,
              be2_ref, w3_ref, b3_ref, o_ref):
    def bn(t, gm, bt):
        mu = jnp.mean(t, axis=0, keepdims=True)
        var = jnp.mean(t * t, axis=0, keepdims=True) - mu * mu
        return gm * (t - mu) / jnp.sqrt(var + 1e-5) + bt

    t = _dotd(g_ref[...], w1_ref[...]) + b1_ref[...]
    t = jnp.maximum(bn(t, g1_ref[...], be1_ref[...]), 0.0)
    t = _dotd(t, w2_ref[...]) + b2_ref[...]
    t = jnp.maximum(bn(t, g2_ref[...], be2_ref[...]), 0.0)
    o_ref[...] = _dotd(t, w3_ref[...]) + b3_ref[...]


def _mlp(g, mp):
    args = [g, mp['W1'], mp['b1'].reshape(1, -1), mp['g1'].reshape(1, -1),
            mp['be1'].reshape(1, -1), mp['W2'], mp['b2'].reshape(1, -1),
            mp['g2'].reshape(1, -1), mp['be2'].reshape(1, -1), mp['W3'],
            mp['b3'].reshape(1, -1)]
    return pl.pallas_call(
        _mlp_body,
        out_shape=jax.ShapeDtypeStruct((NUM_GRAPHS, 3), f32),
    )(*args)


def kernel(x, edge_index, edge_attr, batch, params):
    xi = x.astype(i32)
    src = edge_index[0].astype(i32)
    dst = edge_index[1].astype(i32)
    ea = edge_attr.astype(i32)
    cls = ea[:, 0] * 4 + ea[:, 1] * 2 + ea[:, 2]

    keys = jnp.sort((dst << 17) | (src << 3) | cls)
    s_dst = keys >> 17
    bounds = jnp.arange(0, NW + 1, dtype=i32) * NPT
    ptr = jnp.searchsorted(s_dst, jnp.arange(N + 1, dtype=i32)).astype(i32)
    tb = jnp.searchsorted(s_dst, bounds).astype(i32)
    lo = tb[:-1] // CK
    hi = (tb[1:] + (CK - 1)) // CK
    meta = jnp.concatenate([lo, hi, jnp.zeros(2 * NW, i32)]).astype(i32)

    plo = jnp.pad(ptr[:N], (0, NPAD - N)).reshape(NPAD // H, H)
    phi = jnp.pad(ptr[1:N + 1], (0, NPAD - N)).reshape(NPAD // H, H)

    ae = params['atom_emb']
    base = sum(a[0] for a in ae).reshape(1, H)
    dmat = jnp.concatenate(
        [jnp.stack([a[1] - a[0] for a in ae], axis=0),
         jnp.zeros((H - len(ae), H), f32)], axis=0)
    xf = jnp.pad(xi.astype(f32), ((0, NPAD - N), (0, H - xi.shape[1])))
    h_pad = _enc(xf, dmat, base)
    inv_rs, sc1_rs, sc2_rs = _cntk(plo, phi)
    bcast = lambda t: jnp.broadcast_to(t.reshape(NPAD, 1), (NPAD, H))
    invb, sc1b, sc2b = bcast(inv_rs), bcast(sc1_rs), bcast(sc2_rs)

    be = params['bond_emb']
    bits = np.array([[(c >> 2) & 1, (c >> 1) & 1, c & 1] for c in range(8)])
    etab = (be[0][bits[:, 0]] + be[1][bits[:, 1]] + be[2][bits[:, 2]])
    etab = jnp.concatenate([etab, jnp.tile(etab[:1], (8, 1))], axis=0)

    for cp in params['convs']:
        wpre = cp['W_pre']
        a, bt, selfv, selfv2, ctab = _pre(
            h_pad, wpre[:H], wpre[H:2 * H], cp['W_edge'], wpre[2 * H:],
            cp['b_pre'].reshape(1, H), cp['b_edge'].reshape(1, H), etab)
        bt2 = bt.reshape(NPAD // 2, 2 * H)
        s1, s2, mn, mx = _sc_edge(keys, bt2, ctab, meta,
                                  selfv.reshape(NPAD // 2, 2 * H),
                                  selfv2.reshape(NPAD // 2, 2 * H))
        s1, s2, mn, mx = (t.reshape(NPAD, H) for t in (s1, s2, mn, mx))
        outp = _comb(h_pad, a, s1, s2, mn, mx, invb, sc1b, sc2b, cp['W_post'],
                     cp['b_post'].reshape(1, H), cp['W_lin'],
                     cp['b_lin'].reshape(1, H))
        h = _bn_relu(outp[:N], cp['bn_g'].reshape(1, H), cp['bn_b'].reshape(1, H))
        h_pad = jnp.pad(h, ((0, NPAD - N), (0, 0)))

    bi = batch.astype(i32)
    onehot = (bi[:, None] == jnp.arange(NUM_GRAPHS, dtype=i32)[None, :]).astype(f32)
    g = _pool(onehot, h)
    return _mlp(g, params['mlp'])
